# baseline (speedup 1.0000x reference)
import functools

import jax
import jax.numpy as jnp
from jax import lax
from jax.experimental import pallas as pl
from jax.experimental.pallas import tpu as pltpu
from jax.experimental.pallas import tpu_sc as plsc

L_SEQ = 50
B = 4096
D = 64
NC = 2
NS = 16
NW = NC * NS
BPW = B // NW
NBUF = 4


def _sc_body(title_hbm, table_hbm, sums_hbm, idx_v, dst_v, bufs, acc_sh, *sems):
  gsems = sems[:NBUF]
  ssems = sems[NBUF:]
  c = lax.axis_index("c")
  s = lax.axis_index("s")
  wid = s * NC + c
  base = wid * BPW
  local = s * BPW

  pltpu.sync_copy(title_hbm.at[:, pl.ds(base, BPW)], idx_v)

  for c8 in range(BPW // 16):
    dst_v[pl.ds(c8 * 16, 16)] = (
        local + c8 * 16 + lax.broadcasted_iota(jnp.int32, (16,), 0)
    )

  gdesc = {}
  sdesc = {}

  def gstart(l):
    b = l % NBUF
    d = pltpu.make_async_copy(table_hbm.at[idx_v.at[l]], bufs.at[b], gsems[b])
    d.start()
    gdesc[l] = d

  def sstart(l):
    b = l % NBUF
    d = pltpu.make_async_copy(bufs.at[b], acc_sh.at[dst_v], ssems[b])
    d.start(add=True)
    sdesc[l] = d

  for l in range(NBUF):
    gstart(l)
  gdesc[0].wait()
  init = pltpu.make_async_copy(bufs.at[0], acc_sh.at[pl.ds(local, BPW)],
                               ssems[0])
  init.start()
  init.wait()
  gstart(NBUF)

  for l in range(1, L_SEQ):
    gdesc[l].wait()
    sstart(l)
    if l >= 2:
      sdesc[l - 1].wait()
      nl = l - 1 + NBUF
      if nl < L_SEQ:
        gstart(nl)
  sdesc[L_SEQ - 1].wait()

  pltpu.sync_copy(acc_sh.at[pl.ds(local, BPW)],
                  sums_hbm.at[pl.ds(base, BPW)])


_sc_sums = functools.partial(
    pl.kernel,
    out_type=jax.ShapeDtypeStruct((B, D), jnp.float32),
    mesh=plsc.VectorSubcoreMesh(core_axis_name="c", subcore_axis_name="s"),
    scratch_types=[
        pltpu.VMEM((L_SEQ, BPW), jnp.int32),
        pltpu.VMEM((BPW,), jnp.int32),
        pltpu.VMEM((NBUF, BPW, D), jnp.float32),
        pltpu.VMEM_SHARED((NS * BPW, D), jnp.float32),
    ] + [pltpu.SemaphoreType.DMA] * (2 * NBUF),
    compiler_params=pltpu.CompilerParams(use_tc_tiling_on_sc=False),
)(_sc_body)


def _tc_body(sums_ref, ctx_ref, w_ref, b_ref, out_ref):
  w = w_ref[...]
  w64 = w[:, :D]
  wctx = w[:, D:]
  acc = jnp.sum(sums_ref[...] * w64, axis=1)
  ctxdot = jnp.sum(ctx_ref[...] * wctx, axis=1)
  out_ref[...] = acc * (1.0 / L_SEQ) + ctxdot + b_ref[0]


def _tc_epilogue(sums, ctx, fc_w, fc_b):
  return pl.pallas_call(
      _tc_body,
      out_shape=jax.ShapeDtypeStruct((B,), jnp.float32),
      in_specs=[
          pl.BlockSpec(memory_space=pltpu.VMEM),
          pl.BlockSpec(memory_space=pltpu.VMEM),
          pl.BlockSpec(memory_space=pltpu.VMEM),
          pl.BlockSpec(memory_space=pltpu.SMEM),
      ],
      out_specs=pl.BlockSpec(memory_space=pltpu.VMEM),
  )(sums, ctx, fc_w, fc_b)


@jax.jit
def kernel(title, serious, spoiler, nsfw, num_comments, emb_table, fc_w, fc_b):
  title = title.astype(jnp.int32)
  sums = _sc_sums(title, emb_table)
  ctx = jnp.stack([serious, spoiler, nsfw, num_comments],
                  axis=1).astype(jnp.float32)
  return _tc_epilogue(sums, ctx, fc_w, fc_b)
